# baseline (speedup 1.0000x reference)
import functools

import numpy as np
import jax
import jax.numpy as jnp
from jax import lax
from jax.experimental import pallas as pl
from jax.experimental.pallas import tpu as pltpu
from jax.experimental.pallas import tpu_sc as plsc

_ALPHA = 4
_NBUF = 4


def _sc_body(src_offs, row_words, chunk, nc, fr, out, b0, b1, b2, b3,
             l0, l1, l2, l3, s0, s1, s2, s3):
    bufs = (b0, b1, b2, b3)
    lsem = (l0, l1, l2, l3)
    ssem = (s0, s1, s2, s3)
    nrows = len(src_offs)
    wid = lax.axis_index("s") * nc + lax.axis_index("c")
    base = wid * chunk

    def mk_load(j):
        b = j % _NBUF
        return pltpu.make_async_copy(
            fr.at[pl.ds(src_offs[j] + base, chunk)], bufs[b], lsem[b])

    def mk_store(j):
        b = j % _NBUF
        return pltpu.make_async_copy(
            bufs[b], out.at[pl.ds(j * row_words + base, chunk)], ssem[b])

    for k in range(min(_NBUF, nrows)):
        mk_load(k).start()
    for j in range(nrows):
        mk_load(j).wait()
        mk_store(j).start()
        if j + _NBUF < nrows:
            mk_store(j).wait()
            mk_load(j + _NBUF).start()
    for j in range(max(nrows - _NBUF, 0), nrows):
        mk_store(j).wait()


def kernel(frames):
    C, T, H, W = frames.shape
    Ts = T // _ALPHA
    row_words = H * W
    idx = np.linspace(0.0, T - 1, Ts).astype(np.int32)
    src_rows = (np.arange(C)[:, None] * T + idx[None, :]).reshape(-1)
    src_offs = [int(r) * row_words for r in src_rows]

    info = plsc.get_sparse_core_info()
    nc, ns = info.num_cores, info.num_subcores
    nw = nc * ns
    chunk = row_words // nw

    mesh = plsc.VectorSubcoreMesh(core_axis_name="c", subcore_axis_name="s")
    sc_copy = pl.kernel(
        functools.partial(_sc_body, src_offs, row_words, chunk, nc),
        out_type=jax.ShapeDtypeStruct((C * Ts * row_words,), frames.dtype),
        mesh=mesh,
        scratch_types=(
            [pltpu.VMEM((chunk,), frames.dtype) for _ in range(_NBUF)]
            + [pltpu.SemaphoreType.DMA] * (2 * _NBUF)
        ),
    )
    slow = sc_copy(frames.reshape(-1))
    return (slow.reshape(C, Ts, H, W), frames)


# speedup vs baseline: 1.0093x; 1.0093x over previous
import functools

import numpy as np
import jax
import jax.numpy as jnp
from jax import lax
from jax.experimental import pallas as pl
from jax.experimental.pallas import tpu as pltpu
from jax.experimental.pallas import tpu_sc as plsc

_ALPHA = 4
_NBUF = 3
_PARTS = 8


def _sc_body(src_rows, row_words, chunk, nc, nw, fr, out, b0, b1, b2,
             l0, l1, l2, s0, s1, s2):
    bufs = (b0, b1, b2)
    lsem = (l0, l1, l2)
    ssem = (s0, s1, s2)
    nsteps = len(src_rows) * _PARTS // nw
    rows_per_step = nw // _PARTS
    wid = lax.axis_index("s") * nc + lax.axis_index("c")
    row_sel = lax.shift_right_logical(wid, 3)
    part_off = (wid & (_PARTS - 1)) * chunk
    dst_base = wid * chunk

    def mk_load(k):
        b = k % _NBUF
        row = jnp.int32(k * rows_per_step) + row_sel
        c = lax.shift_right_logical(row, 4)
        t = row & 15
        src_row = c * 64 + lax.shift_right_logical(t * 1101005, 18)
        src = src_row * row_words + part_off
        return pltpu.make_async_copy(fr.at[pl.ds(src, chunk)], bufs[b], lsem[b])

    def mk_store(k):
        b = k % _NBUF
        dst = k * nw * chunk + dst_base
        return pltpu.make_async_copy(bufs[b], out.at[pl.ds(dst, chunk)], ssem[b])

    for k in range(min(_NBUF, nsteps)):
        mk_load(k).start()
    for k in range(nsteps):
        mk_load(k).wait()
        mk_store(k).start()
        if k + _NBUF < nsteps:
            mk_store(k).wait()
            mk_load(k + _NBUF).start()
    for k in range(max(nsteps - _NBUF, 0), nsteps):
        mk_store(k).wait()


def kernel(frames):
    C, T, H, W = frames.shape
    Ts = T // _ALPHA
    row_words = H * W
    idx = np.linspace(0.0, T - 1, Ts).astype(np.int32)
    src_rows = [int(r) for r in
                (np.arange(C)[:, None] * T + idx[None, :]).reshape(-1)]

    info = plsc.get_sparse_core_info()
    nc, ns = info.num_cores, info.num_subcores
    nw = nc * ns
    chunk = row_words // _PARTS

    mesh = plsc.VectorSubcoreMesh(core_axis_name="c", subcore_axis_name="s")
    sc_copy = pl.kernel(
        functools.partial(_sc_body, src_rows, row_words, chunk, nc, nw),
        out_type=jax.ShapeDtypeStruct((C * Ts * row_words,), frames.dtype),
        mesh=mesh,
        scratch_types=(
            [pltpu.VMEM((chunk,), frames.dtype) for _ in range(_NBUF)]
            + [pltpu.SemaphoreType.DMA] * (2 * _NBUF)
        ),
    )
    slow = sc_copy(frames.reshape(-1))
    return (slow.reshape(C, Ts, H, W), frames)


# speedup vs baseline: 2.1074x; 2.0879x over previous
import functools

import numpy as np
import jax
import jax.numpy as jnp
from jax import lax
from jax.experimental import pallas as pl
from jax.experimental.pallas import tpu as pltpu
from jax.experimental.pallas import tpu_sc as plsc

_ALPHA = 4
_NBUF = 3
_PARTS = 8


def _sc_body(nrows, band, w, nc, nw, fr, out, b0, b1, b2,
             l0, l1, l2, s0, s1, s2):
    bufs = (b0, b1, b2)
    lsem = (l0, l1, l2)
    ssem = (s0, s1, s2)
    nsteps = nrows * _PARTS // nw
    rows_per_step = nw // _PARTS
    wid = lax.axis_index("s") * nc + lax.axis_index("c")
    row_sel = lax.shift_right_logical(wid, 3)
    band_lo = (wid & (_PARTS - 1)) * band

    def mk_load(k):
        b = k % _NBUF
        row = jnp.int32(k * rows_per_step) + row_sel
        c = lax.shift_right_logical(row, 4)
        t = row & 15
        src_row = c * 64 + lax.shift_right_logical(t * 1101005, 18)
        return pltpu.make_async_copy(
            fr.at[src_row, pl.ds(band_lo, band), :], bufs[b], lsem[b])

    def mk_store(k):
        b = k % _NBUF
        dst_row = jnp.int32(k * rows_per_step) + row_sel
        return pltpu.make_async_copy(
            bufs[b], out.at[dst_row, pl.ds(band_lo, band), :], ssem[b])

    for k in range(min(_NBUF, nsteps)):
        mk_load(k).start()
    for k in range(nsteps):
        mk_load(k).wait()
        mk_store(k).start()
        if k + _NBUF < nsteps:
            mk_store(k).wait()
            mk_load(k + _NBUF).start()
    for k in range(max(nsteps - _NBUF, 0), nsteps):
        mk_store(k).wait()


def kernel(frames):
    C, T, H, W = frames.shape
    Ts = T // _ALPHA
    band = H // _PARTS

    info = plsc.get_sparse_core_info()
    nc, ns = info.num_cores, info.num_subcores
    nw = nc * ns

    mesh = plsc.VectorSubcoreMesh(core_axis_name="c", subcore_axis_name="s")
    sc_copy = pl.kernel(
        functools.partial(_sc_body, C * Ts, band, W, nc, nw),
        out_type=jax.ShapeDtypeStruct((C * Ts, H, W), frames.dtype),
        mesh=mesh,
        scratch_types=(
            [pltpu.VMEM((band, W), frames.dtype) for _ in range(_NBUF)]
            + [pltpu.SemaphoreType.DMA] * (2 * _NBUF)
        ),
        compiler_params=pltpu.CompilerParams(use_tc_tiling_on_sc=True),
    )
    slow = sc_copy(frames.reshape(C * T, H, W))
    return (slow.reshape(C, Ts, H, W), frames)


# speedup vs baseline: 2.3047x; 1.0936x over previous
import functools

import numpy as np
import jax
import jax.numpy as jnp
from jax.experimental import pallas as pl
from jax.experimental.pallas import tpu as pltpu

_ALPHA = 4
_K = 8
_PARTS = 2


def _multi_chain_copy(assign, band, fr_ref, out_ref, bufs, lsems, ssems):
    S = len(assign[0])

    def mk_load(c, s):
        srow, _, p = assign[c][s]
        return pltpu.make_async_copy(
            fr_ref.at[srow, pl.ds(p * band, band), :],
            bufs.at[c, s % 2], lsems.at[c, s % 2])

    def mk_store(c, s):
        _, drow, p = assign[c][s]
        return pltpu.make_async_copy(
            bufs.at[c, s % 2],
            out_ref.at[drow, pl.ds(p * band, band), :], ssems.at[c, s % 2])

    for c in range(_K):
        mk_load(c, 0).start()
    for s in range(S):
        for c in range(_K):
            mk_load(c, s).wait()
            mk_store(c, s).start()
        if s + 1 < S:
            for c in range(_K):
                if s >= 1:
                    mk_store(c, s - 1).wait()
                mk_load(c, s + 1).start()
    for c in range(_K):
        if S >= 2:
            mk_store(c, S - 2).wait()
        mk_store(c, S - 1).wait()


def kernel(frames):
    C, T, H, W = frames.shape
    Ts = T // _ALPHA
    band = H // _PARTS
    idx = np.linspace(0.0, T - 1, Ts).astype(np.int32)
    src_rows = [int(r) for r in
                (np.arange(C)[:, None] * T + idx[None, :]).reshape(-1)]

    nchunks = C * Ts * _PARTS
    assert nchunks % _K == 0
    assign = [[] for _ in range(_K)]
    for g in range(nchunks):
        assign[g % _K].append((src_rows[g // _PARTS], g // _PARTS, g % _PARTS))

    slow = pl.pallas_call(
        functools.partial(_multi_chain_copy, assign, band),
        out_shape=jax.ShapeDtypeStruct((C * Ts, H, W), frames.dtype),
        in_specs=[pl.BlockSpec(memory_space=pltpu.HBM)],
        out_specs=pl.BlockSpec(memory_space=pltpu.HBM),
        scratch_shapes=[
            pltpu.VMEM((_K, 2, band, W), frames.dtype),
            pltpu.SemaphoreType.DMA((_K, 2)),
            pltpu.SemaphoreType.DMA((_K, 2)),
        ],
    )(frames.reshape(C * T, H, W))
    return (slow.reshape(C, Ts, H, W), frames)


# speedup vs baseline: 2.3052x; 1.0002x over previous
import functools

import numpy as np
import jax
import jax.numpy as jnp
from jax.experimental import pallas as pl
from jax.experimental.pallas import tpu as pltpu

_ALPHA = 4
_K = 16
_PARTS = 4


def _multi_chain_copy(assign, band, fr_ref, out_ref, bufs, lsems, ssems):
    S = len(assign[0])

    def mk_load(c, s):
        srow, _, p = assign[c][s]
        return pltpu.make_async_copy(
            fr_ref.at[srow, pl.ds(p * band, band), :],
            bufs.at[c, s % 2], lsems.at[c, s % 2])

    def mk_store(c, s):
        _, drow, p = assign[c][s]
        return pltpu.make_async_copy(
            bufs.at[c, s % 2],
            out_ref.at[drow, pl.ds(p * band, band), :], ssems.at[c, s % 2])

    for c in range(_K):
        mk_load(c, 0).start()
    for s in range(S):
        for c in range(_K):
            mk_load(c, s).wait()
            mk_store(c, s).start()
        if s + 1 < S:
            for c in range(_K):
                if s >= 1:
                    mk_store(c, s - 1).wait()
                mk_load(c, s + 1).start()
    for c in range(_K):
        if S >= 2:
            mk_store(c, S - 2).wait()
        mk_store(c, S - 1).wait()


def kernel(frames):
    C, T, H, W = frames.shape
    Ts = T // _ALPHA
    band = H // _PARTS
    idx = np.linspace(0.0, T - 1, Ts).astype(np.int32)
    src_rows = [int(r) for r in
                (np.arange(C)[:, None] * T + idx[None, :]).reshape(-1)]

    nchunks = C * Ts * _PARTS
    assert nchunks % _K == 0
    assign = [[] for _ in range(_K)]
    for g in range(nchunks):
        assign[g % _K].append((src_rows[g // _PARTS], g // _PARTS, g % _PARTS))

    slow = pl.pallas_call(
        functools.partial(_multi_chain_copy, assign, band),
        out_shape=jax.ShapeDtypeStruct((C * Ts, H, W), frames.dtype),
        in_specs=[pl.BlockSpec(memory_space=pltpu.HBM)],
        out_specs=pl.BlockSpec(memory_space=pltpu.HBM),
        scratch_shapes=[
            pltpu.VMEM((_K, 2, band, W), frames.dtype),
            pltpu.SemaphoreType.DMA((_K, 2)),
            pltpu.SemaphoreType.DMA((_K, 2)),
        ],
    )(frames.reshape(C * T, H, W))
    return (slow.reshape(C, Ts, H, W), frames)
